# baseline (speedup 1.0000x reference)
import jax
import jax.numpy as jnp
from jax import lax
from jax.experimental import pallas as pl
from jax.experimental.pallas import tpu as pltpu
from jax.experimental.pallas import tpu_sc as plsc
import functools

Nb, Nt, E, D = 10000, 2000, 320000, 128
NC, NS = 2, 16
NW = NC * NS
C = 80
EW = E // NW
CH = EW // C
SLAB = 128
G = 25
NSC = CH // G
ZR = 64
NTP = Nt
TREM = Nt - (NS - 1) * SLAB
NBP = NS * 5 * SLAB


def _seg_body(nrows, table_hbm, gidx_hbm, sidx4d_hbm, sum_out, cnt_out,
              acc, gi_v, si2d, rows0, rows1, hist, zbuf,
              sem0, sem1, sem2, sem3):
    c = lax.axis_index("c")
    s = lax.axis_index("s")
    wid = c * NS + s
    base_e = wid * EW

    full = nrows // SLAB
    rem = nrows - full * SLAB
    spt = full // NS if rem == 0 else 1

    zeros16 = jnp.zeros((16,), jnp.float32)
    ones16 = jnp.ones((16,), jnp.float32)

    def _fill(i, _):
        for j in range(D // 16):
            zbuf[i, pl.ds(j * 16, 16)] = zeros16
        return 0
    lax.fori_loop(0, ZR, _fill, 0)

    def _fill_hist(i, _):
        hist[pl.ds(i * 16, 16)] = zeros16
        return 0
    lax.fori_loop(0, nrows // 16, _fill_hist, 0)

    if rem == 0:
        for j in range(spt):
            off = (s * spt + j) * SLAB
            for z in range(SLAB // ZR):
                pltpu.sync_copy(zbuf, acc.at[pl.ds(off + z * ZR, ZR)])
    else:
        @pl.when(s < full)
        def _():
            for z in range(SLAB // ZR):
                pltpu.sync_copy(zbuf, acc.at[pl.ds(s * SLAB + z * ZR, ZR)])

        @pl.when(s == full)
        def _():
            pltpu.sync_copy(zbuf, acc.at[pl.ds(full * SLAB, ZR)])
            pltpu.sync_copy(zbuf.at[pl.ds(0, rem - ZR)],
                            acc.at[pl.ds(full * SLAB + ZR, rem - ZR)])
    plsc.subcore_barrier()

    def _stage(j, rows, semg):
        pltpu.async_copy(table_hbm.at[gi_v.at[pl.ds(j * C, C)]], rows, semg)
        idxs = si2d.at[j]
        for k in range(C // 16):
            idx16 = idxs[pl.ds(k * 16, 16)]
            plsc.addupdate_scatter(hist, [idx16], ones16)

    def _gwait(j, rows, semg):
        pltpu.make_async_copy(table_hbm.at[gi_v.at[pl.ds(j * C, C)]],
                              rows, semg).wait()

    def _scat(j, rows, sems):
        pltpu.async_copy(rows, acc.at[si2d.at[j]], sems, add=True)

    def _swait(j, rows, sems):
        pltpu.make_async_copy(rows, acc.at[si2d.at[j]], sems).wait()

    assert G % 2 == 1

    def _super(sc, _):
        pltpu.sync_copy(gidx_hbm.at[pl.ds(base_e + sc * G * C, G * C)], gi_v)
        pltpu.sync_copy(sidx4d_hbm.at[wid, sc], si2d)
        _stage(0, rows0, sem0)

        def _pair(p, _):
            _stage(2 * p + 1, rows1, sem1)
            _gwait(2 * p, rows0, sem0)
            _scat(2 * p, rows0, sem2)
            _gwait(2 * p + 1, rows1, sem1)
            _scat(2 * p + 1, rows1, sem3)
            _swait(2 * p, rows0, sem2)
            _stage(2 * p + 2, rows0, sem0)
            _swait(2 * p + 1, rows1, sem3)
            return 0
        lax.fori_loop(0, (G - 1) // 2, _pair, 0)
        _gwait(G - 1, rows0, sem0)
        pltpu.sync_copy(rows0, acc.at[si2d.at[G - 1]], add=True)
        return 0
    lax.fori_loop(0, NSC, _super, 0)
    plsc.subcore_barrier()

    def _out_piece(off, n):
        pltpu.sync_copy(acc.at[pl.ds(off, n)], zbuf.at[pl.ds(0, n)])
        pltpu.sync_copy(zbuf.at[pl.ds(0, n)],
                        sum_out.at[pl.ds(c * nrows + off, n)])

    if rem == 0:
        for j in range(spt):
            off = (s * spt + j) * SLAB
            for z in range(SLAB // ZR):
                _out_piece(off + z * ZR, ZR)
    else:
        @pl.when(s < full)
        def _():
            for z in range(SLAB // ZR):
                _out_piece(s * SLAB + z * ZR, ZR)

        @pl.when(s == full)
        def _():
            _out_piece(full * SLAB, ZR)
            _out_piece(full * SLAB + ZR, rem - ZR)

    pltpu.sync_copy(hist, cnt_out.at[pl.ds(wid * nrows, nrows)])


def _make_seg(nrows):
    return pl.kernel(
        functools.partial(_seg_body, nrows),
        out_type=[
            jax.ShapeDtypeStruct((NC * nrows, D), jnp.float32),
            jax.ShapeDtypeStruct((NW * nrows,), jnp.float32),
        ],
        mesh=plsc.VectorSubcoreMesh(core_axis_name="c", subcore_axis_name="s"),
        compiler_params=pltpu.CompilerParams(needs_layout_passes=False),
        scratch_types=[
            pltpu.VMEM_SHARED((nrows, D), jnp.float32),
            pltpu.VMEM((G * C,), jnp.int32),
            pltpu.VMEM((G, C), jnp.int32),
            pltpu.VMEM((C, D), jnp.float32),
            pltpu.VMEM((C, D), jnp.float32),
            pltpu.VMEM((nrows,), jnp.float32),
            pltpu.VMEM((ZR, D), jnp.float32),
            pltpu.SemaphoreType.DMA,
            pltpu.SemaphoreType.DMA,
            pltpu.SemaphoreType.DMA,
            pltpu.SemaphoreType.DMA,
        ],
    )


_seg_trait = _make_seg(NTP)
_seg_bact = _make_seg(NBP)


BLK = 400


def _linear_body(x_ref, w_ref, b_ref, o_ref):
    o_ref[...] = jnp.dot(x_ref[...], w_ref[...],
                         preferred_element_type=jnp.float32) + b_ref[...]


def _linear(x, w, b):
    n = x.shape[0]
    return pl.pallas_call(
        _linear_body,
        grid=(n // BLK,),
        in_specs=[
            pl.BlockSpec((BLK, D), lambda i: (i, 0)),
            pl.BlockSpec((D, D), lambda i: (0, 0)),
            pl.BlockSpec((1, D), lambda i: (0, 0)),
        ],
        out_specs=pl.BlockSpec((BLK, D), lambda i: (i, 0)),
        out_shape=jax.ShapeDtypeStruct((n, D), jnp.float32),
    )(x, w, b.reshape(1, D))


def _post_body(t_ref, s0_ref, s1_ref, c_ref, wl_ref, wr_ref,
               b_ref, g_ref, be_ref, h_ref):
    t = t_ref[...]
    ssum = s0_ref[...] + s1_ref[...]
    cnt = jnp.sum(c_ref[...], axis=-1, keepdims=True)
    mean = ssum / jnp.maximum(cnt, 1.0)
    conv = (jnp.dot(mean, wl_ref[...], preferred_element_type=jnp.float32)
            + jnp.dot(t, wr_ref[...], preferred_element_type=jnp.float32)
            + b_ref[...])
    nrm = jnp.sqrt(jnp.sum(conv * conv, axis=-1, keepdims=True))
    conv = conv / jnp.maximum(nrm, 1e-12)
    x = t + conv
    m = jnp.mean(x, axis=-1, keepdims=True)
    v = jnp.mean((x - m) * (x - m), axis=-1, keepdims=True)
    h_ref[...] = (x - m) / jnp.sqrt(v + 1e-5) * g_ref[...] + be_ref[...]


def _post(t, s0, s1, cmat, wl, wr, b, g, be):
    n = t.shape[0]
    return pl.pallas_call(
        _post_body,
        grid=(n // BLK,),
        in_specs=[
            pl.BlockSpec((BLK, D), lambda i: (i, 0)),
            pl.BlockSpec((BLK, D), lambda i: (i, 0)),
            pl.BlockSpec((BLK, D), lambda i: (i, 0)),
            pl.BlockSpec((BLK, NW), lambda i: (i, 0)),
            pl.BlockSpec((D, D), lambda i: (0, 0)),
            pl.BlockSpec((D, D), lambda i: (0, 0)),
            pl.BlockSpec((1, D), lambda i: (0, 0)),
            pl.BlockSpec((1, D), lambda i: (0, 0)),
            pl.BlockSpec((1, D), lambda i: (0, 0)),
        ],
        out_specs=pl.BlockSpec((BLK, D), lambda i: (i, 0)),
        out_shape=jax.ShapeDtypeStruct((n, D), jnp.float32),
    )(t, s0, s1, cmat, wl, wr, b.reshape(1, D), g.reshape(1, D),
      be.reshape(1, D))


def _final_body(a1_ref, a2_ref, ht_ref, hb_ref, w_ref, wmp_ref, bmp_ref,
                g_ref, be_ref, o_ref):
    ht = ht_ref[...]
    a1 = jnp.dot(a1_ref[...], ht, preferred_element_type=jnp.float32)
    a2 = jnp.dot(a2_ref[...], ht, preferred_element_type=jnp.float32)
    wv = w_ref[...]
    mx = jnp.max(wv)
    ev = jnp.exp(wv - mx)
    es = jnp.sum(ev)
    y = hb_ref[...] + (a1 * ev[:, 0:1] + a2 * ev[:, 1:2]) / es
    z = jnp.dot(y, wmp_ref[...], preferred_element_type=jnp.float32) + bmp_ref[...]
    m = jnp.mean(z, axis=-1, keepdims=True)
    v = jnp.mean((z - m) * (z - m), axis=-1, keepdims=True)
    o_ref[...] = (z - m) / jnp.sqrt(v + 1e-5) * g_ref[...] + be_ref[...]


def _final(adj1, adj2, h_t, h_b, mp_w, wmp, bmp, g, be):
    return pl.pallas_call(
        _final_body,
        grid=(Nb // BLK,),
        in_specs=[
            pl.BlockSpec((BLK, Nt), lambda i: (i, 0)),
            pl.BlockSpec((BLK, Nt), lambda i: (i, 0)),
            pl.BlockSpec((Nt, D), lambda i: (0, 0)),
            pl.BlockSpec((BLK, D), lambda i: (i, 0)),
            pl.BlockSpec((1, 2), lambda i: (0, 0)),
            pl.BlockSpec((D, D), lambda i: (0, 0)),
            pl.BlockSpec((1, D), lambda i: (0, 0)),
            pl.BlockSpec((1, D), lambda i: (0, 0)),
            pl.BlockSpec((1, D), lambda i: (0, 0)),
        ],
        out_specs=pl.BlockSpec((BLK, D), lambda i: (i, 0)),
        out_shape=jax.ShapeDtypeStruct((Nb, D), jnp.float32),
    )(adj1, adj2, h_t, h_b, mp_w.reshape(1, 2), wmp, bmp.reshape(1, D),
      g.reshape(1, D), be.reshape(1, D))


def kernel(x_bacteria, x_trait, adj1, adj2, Wb, bb, Wt, bt, Wl_bt, Wr_bt,
           b_bt, Wl_tb, Wr_tb, b_tb, g_b, be_b, g_t, be_t, mp_w, Wmp, bmp,
           g_mp, be_mp, edge_src, edge_dst):
    tb = _linear(x_bacteria, Wb, bb)
    tt = _linear(x_trait, Wt, bt)

    src4d = edge_src.reshape(NW, NSC, G, C)
    dst4d = edge_dst.reshape(NW, NSC, G, C)
    sum_bt, cnt_bt = _seg_trait(tb, edge_src, dst4d)
    sum_tb, cnt_tb = _seg_bact(tt, edge_dst, src4d)

    h_t = _post(tt, sum_bt[:Nt], sum_bt[NTP:NTP + Nt],
                cnt_bt.reshape(NW, NTP).T[:Nt],
                Wl_bt, Wr_bt, b_bt, g_t, be_t)
    h_b = _post(tb, sum_tb[:Nb], sum_tb[NBP:NBP + Nb],
                cnt_tb.reshape(NW, NBP).T[:Nb],
                Wl_tb, Wr_tb, b_tb, g_b, be_b)

    return _final(adj1, adj2, h_t, h_b, mp_w, Wmp, bmp, g_mp, be_mp)


# speedup vs baseline: 1.0958x; 1.0958x over previous
import jax
import jax.numpy as jnp
from jax import lax
from jax.experimental import pallas as pl
from jax.experimental.pallas import tpu as pltpu
from jax.experimental.pallas import tpu_sc as plsc
import functools

Nb, Nt, E, D = 10000, 2000, 320000, 128
NC, NS = 2, 16
NW = NC * NS
C = 80
EW = E // NW
CH = EW // C
SLAB = 128
G = 25
NSC = CH // G
ZR = 64
NTP = Nt
TREM = Nt - (NS - 1) * SLAB
NBP = NS * 5 * SLAB


def _seg_body(nrows, table_hbm, gidx_hbm, sidx4d_hbm, sum_out, cnt_out,
              acc, gi_v, si2d, rows0, rows1, hist, zbuf,
              sem0, sem1):
    c = lax.axis_index("c")
    s = lax.axis_index("s")
    wid = c * NS + s
    base_e = wid * EW

    full = nrows // SLAB
    rem = nrows - full * SLAB
    spt = full // NS if rem == 0 else 1

    zeros16 = jnp.zeros((16,), jnp.float32)
    ones16 = jnp.ones((16,), jnp.float32)

    def _fill(i, _):
        for j in range(D // 16):
            zbuf[i, pl.ds(j * 16, 16)] = zeros16
        return 0
    lax.fori_loop(0, ZR, _fill, 0)

    def _fill_hist(i, _):
        hist[pl.ds(i * 16, 16)] = zeros16
        return 0
    lax.fori_loop(0, nrows // 16, _fill_hist, 0)

    if rem == 0:
        for j in range(spt):
            off = (s * spt + j) * SLAB
            for z in range(SLAB // ZR):
                pltpu.sync_copy(zbuf, acc.at[pl.ds(off + z * ZR, ZR)])
    else:
        @pl.when(s < full)
        def _():
            for z in range(SLAB // ZR):
                pltpu.sync_copy(zbuf, acc.at[pl.ds(s * SLAB + z * ZR, ZR)])

        @pl.when(s == full)
        def _():
            pltpu.sync_copy(zbuf, acc.at[pl.ds(full * SLAB, ZR)])
            pltpu.sync_copy(zbuf.at[pl.ds(0, rem - ZR)],
                            acc.at[pl.ds(full * SLAB + ZR, rem - ZR)])
    plsc.subcore_barrier()

    def _stage(j, rows, semg):
        pltpu.async_copy(table_hbm.at[gi_v.at[pl.ds(j * C, C)]], rows, semg)
        idxs = si2d.at[j]
        for k in range(C // 16):
            idx16 = idxs[pl.ds(k * 16, 16)]
            plsc.addupdate_scatter(hist, [idx16], ones16)

    def _complete(j, rows, semg):
        pltpu.make_async_copy(table_hbm.at[gi_v.at[pl.ds(j * C, C)]],
                              rows, semg).wait()
        pltpu.sync_copy(rows, acc.at[si2d.at[j]], add=True)

    assert G % 2 == 1

    def _super(sc, _):
        pltpu.sync_copy(gidx_hbm.at[pl.ds(base_e + sc * G * C, G * C)], gi_v)
        pltpu.sync_copy(sidx4d_hbm.at[wid, sc], si2d)
        _stage(0, rows0, sem0)

        def _pair(p, _):
            _stage(2 * p + 1, rows1, sem1)
            _complete(2 * p, rows0, sem0)
            _stage(2 * p + 2, rows0, sem0)
            _complete(2 * p + 1, rows1, sem1)
            return 0
        lax.fori_loop(0, (G - 1) // 2, _pair, 0)
        _complete(G - 1, rows0, sem0)
        return 0
    lax.fori_loop(0, NSC, _super, 0)
    plsc.subcore_barrier()

    def _out_piece(off, n):
        pltpu.sync_copy(acc.at[pl.ds(off, n)], zbuf.at[pl.ds(0, n)])
        pltpu.sync_copy(zbuf.at[pl.ds(0, n)],
                        sum_out.at[pl.ds(c * nrows + off, n)])

    if rem == 0:
        for j in range(spt):
            off = (s * spt + j) * SLAB
            for z in range(SLAB // ZR):
                _out_piece(off + z * ZR, ZR)
    else:
        @pl.when(s < full)
        def _():
            for z in range(SLAB // ZR):
                _out_piece(s * SLAB + z * ZR, ZR)

        @pl.when(s == full)
        def _():
            _out_piece(full * SLAB, ZR)
            _out_piece(full * SLAB + ZR, rem - ZR)

    pltpu.sync_copy(hist, cnt_out.at[pl.ds(wid * nrows, nrows)])


def _make_seg(nrows):
    return pl.kernel(
        functools.partial(_seg_body, nrows),
        out_type=[
            jax.ShapeDtypeStruct((NC * nrows, D), jnp.float32),
            jax.ShapeDtypeStruct((NW * nrows,), jnp.float32),
        ],
        mesh=plsc.VectorSubcoreMesh(core_axis_name="c", subcore_axis_name="s"),
        compiler_params=pltpu.CompilerParams(needs_layout_passes=False),
        scratch_types=[
            pltpu.VMEM_SHARED((nrows, D), jnp.float32),
            pltpu.VMEM((G * C,), jnp.int32),
            pltpu.VMEM((G, C), jnp.int32),
            pltpu.VMEM((C, D), jnp.float32),
            pltpu.VMEM((C, D), jnp.float32),
            pltpu.VMEM((nrows,), jnp.float32),
            pltpu.VMEM((ZR, D), jnp.float32),
            pltpu.SemaphoreType.DMA,
            pltpu.SemaphoreType.DMA,
        ],
    )


_seg_trait = _make_seg(NTP)
_seg_bact = _make_seg(NBP)


BLK = 400


def _linear_body(x_ref, w_ref, b_ref, o_ref):
    o_ref[...] = jnp.dot(x_ref[...], w_ref[...],
                         preferred_element_type=jnp.float32) + b_ref[...]


def _linear(x, w, b):
    n = x.shape[0]
    return pl.pallas_call(
        _linear_body,
        grid=(n // BLK,),
        in_specs=[
            pl.BlockSpec((BLK, D), lambda i: (i, 0)),
            pl.BlockSpec((D, D), lambda i: (0, 0)),
            pl.BlockSpec((1, D), lambda i: (0, 0)),
        ],
        out_specs=pl.BlockSpec((BLK, D), lambda i: (i, 0)),
        out_shape=jax.ShapeDtypeStruct((n, D), jnp.float32),
    )(x, w, b.reshape(1, D))


def _post_body(t_ref, s0_ref, s1_ref, c_ref, wl_ref, wr_ref,
               b_ref, g_ref, be_ref, h_ref):
    t = t_ref[...]
    ssum = s0_ref[...] + s1_ref[...]
    cnt = jnp.sum(c_ref[...], axis=-1, keepdims=True)
    mean = ssum / jnp.maximum(cnt, 1.0)
    conv = (jnp.dot(mean, wl_ref[...], preferred_element_type=jnp.float32)
            + jnp.dot(t, wr_ref[...], preferred_element_type=jnp.float32)
            + b_ref[...])
    nrm = jnp.sqrt(jnp.sum(conv * conv, axis=-1, keepdims=True))
    conv = conv / jnp.maximum(nrm, 1e-12)
    x = t + conv
    m = jnp.mean(x, axis=-1, keepdims=True)
    v = jnp.mean((x - m) * (x - m), axis=-1, keepdims=True)
    h_ref[...] = (x - m) / jnp.sqrt(v + 1e-5) * g_ref[...] + be_ref[...]


def _post(t, s0, s1, cmat, wl, wr, b, g, be):
    n = t.shape[0]
    return pl.pallas_call(
        _post_body,
        grid=(n // BLK,),
        in_specs=[
            pl.BlockSpec((BLK, D), lambda i: (i, 0)),
            pl.BlockSpec((BLK, D), lambda i: (i, 0)),
            pl.BlockSpec((BLK, D), lambda i: (i, 0)),
            pl.BlockSpec((BLK, NW), lambda i: (i, 0)),
            pl.BlockSpec((D, D), lambda i: (0, 0)),
            pl.BlockSpec((D, D), lambda i: (0, 0)),
            pl.BlockSpec((1, D), lambda i: (0, 0)),
            pl.BlockSpec((1, D), lambda i: (0, 0)),
            pl.BlockSpec((1, D), lambda i: (0, 0)),
        ],
        out_specs=pl.BlockSpec((BLK, D), lambda i: (i, 0)),
        out_shape=jax.ShapeDtypeStruct((n, D), jnp.float32),
    )(t, s0, s1, cmat, wl, wr, b.reshape(1, D), g.reshape(1, D),
      be.reshape(1, D))


def _agg_body(a1_ref, a2_ref, ht_ref, w_ref, o_ref):
    ht = ht_ref[...]
    a1 = jnp.dot(a1_ref[...], ht, preferred_element_type=jnp.float32)
    a2 = jnp.dot(a2_ref[...], ht, preferred_element_type=jnp.float32)
    wv = w_ref[...]
    mx = jnp.max(wv)
    ev = jnp.exp(wv - mx)
    es = jnp.sum(ev)
    o_ref[...] = (a1 * ev[:, 0:1] + a2 * ev[:, 1:2]) / es


def _agg(adj1, adj2, h_t, mp_w):
    return pl.pallas_call(
        _agg_body,
        grid=(Nb // BLK,),
        in_specs=[
            pl.BlockSpec((BLK, Nt), lambda i: (i, 0)),
            pl.BlockSpec((BLK, Nt), lambda i: (i, 0)),
            pl.BlockSpec((Nt, D), lambda i: (0, 0)),
            pl.BlockSpec((1, 2), lambda i: (0, 0)),
        ],
        out_specs=pl.BlockSpec((BLK, D), lambda i: (i, 0)),
        out_shape=jax.ShapeDtypeStruct((Nb, D), jnp.float32),
    )(adj1, adj2, h_t, mp_w.reshape(1, 2))


def _out_body(t_ref, s0_ref, s1_ref, c_ref, agg_ref, wl_ref, wr_ref, b_ref,
              g_ref, be_ref, wmp_ref, bmp_ref, gmp_ref, bemp_ref, o_ref):
    t = t_ref[...]
    ssum = s0_ref[...] + s1_ref[...]
    cnt = jnp.sum(c_ref[...], axis=-1, keepdims=True)
    mean = ssum / jnp.maximum(cnt, 1.0)
    conv = (jnp.dot(mean, wl_ref[...], preferred_element_type=jnp.float32)
            + jnp.dot(t, wr_ref[...], preferred_element_type=jnp.float32)
            + b_ref[...])
    nrm = jnp.sqrt(jnp.sum(conv * conv, axis=-1, keepdims=True))
    conv = conv / jnp.maximum(nrm, 1e-12)
    x = t + conv
    m = jnp.mean(x, axis=-1, keepdims=True)
    v = jnp.mean((x - m) * (x - m), axis=-1, keepdims=True)
    h_b = (x - m) / jnp.sqrt(v + 1e-5) * g_ref[...] + be_ref[...]
    y = h_b + agg_ref[...]
    z = (jnp.dot(y, wmp_ref[...], preferred_element_type=jnp.float32)
         + bmp_ref[...])
    m2 = jnp.mean(z, axis=-1, keepdims=True)
    v2 = jnp.mean((z - m2) * (z - m2), axis=-1, keepdims=True)
    o_ref[...] = (z - m2) / jnp.sqrt(v2 + 1e-5) * gmp_ref[...] + bemp_ref[...]


def _out(tb, s0, s1, cmat, agg, wl, wr, b, g, be, wmp, bmp, gmp, bemp):
    vec = pl.BlockSpec((1, D), lambda i: (0, 0))
    return pl.pallas_call(
        _out_body,
        grid=(Nb // BLK,),
        in_specs=[
            pl.BlockSpec((BLK, D), lambda i: (i, 0)),
            pl.BlockSpec((BLK, D), lambda i: (i, 0)),
            pl.BlockSpec((BLK, D), lambda i: (i, 0)),
            pl.BlockSpec((BLK, NW), lambda i: (i, 0)),
            pl.BlockSpec((BLK, D), lambda i: (i, 0)),
            pl.BlockSpec((D, D), lambda i: (0, 0)),
            pl.BlockSpec((D, D), lambda i: (0, 0)),
            vec, vec, vec,
            pl.BlockSpec((D, D), lambda i: (0, 0)),
            vec, vec, vec,
        ],
        out_specs=pl.BlockSpec((BLK, D), lambda i: (i, 0)),
        out_shape=jax.ShapeDtypeStruct((Nb, D), jnp.float32),
    )(tb, s0, s1, cmat, agg, wl, wr, b.reshape(1, D), g.reshape(1, D),
      be.reshape(1, D), wmp, bmp.reshape(1, D), gmp.reshape(1, D),
      bemp.reshape(1, D))


def kernel(x_bacteria, x_trait, adj1, adj2, Wb, bb, Wt, bt, Wl_bt, Wr_bt,
           b_bt, Wl_tb, Wr_tb, b_tb, g_b, be_b, g_t, be_t, mp_w, Wmp, bmp,
           g_mp, be_mp, edge_src, edge_dst):
    tb = _linear(x_bacteria, Wb, bb)
    tt = _linear(x_trait, Wt, bt)

    src4d = edge_src.reshape(NW, NSC, G, C)
    dst4d = edge_dst.reshape(NW, NSC, G, C)
    sum_bt, cnt_bt = _seg_trait(tb, edge_src, dst4d)
    sum_tb, cnt_tb = _seg_bact(tt, edge_dst, src4d)

    h_t = _post(tt, sum_bt[:Nt], sum_bt[NTP:NTP + Nt],
                cnt_bt.reshape(NW, NTP).T[:Nt],
                Wl_bt, Wr_bt, b_bt, g_t, be_t)
    agg = _agg(adj1, adj2, h_t, mp_w)

    return _out(tb, sum_tb[:Nb], sum_tb[NBP:NBP + Nb],
                cnt_tb.reshape(NW, NBP).T[:Nb], agg,
                Wl_tb, Wr_tb, b_tb, g_b, be_b, Wmp, bmp, g_mp, be_mp)


# speedup vs baseline: 1.0963x; 1.0004x over previous
import jax
import jax.numpy as jnp
from jax import lax
from jax.experimental import pallas as pl
from jax.experimental.pallas import tpu as pltpu
from jax.experimental.pallas import tpu_sc as plsc
import functools

Nb, Nt, E, D = 10000, 2000, 320000, 128
NC, NS = 2, 16
NW = NC * NS
C = 80
EW = E // NW
CH = EW // C
SLAB = 128
G = 25
NSC = CH // G
ZR = 64
NTP = Nt
TREM = Nt - (NS - 1) * SLAB
NBP = NS * 5 * SLAB


def _seg_body(nrows, table_hbm, gidx_hbm, sidx4d_hbm, sum_out, cnt_out,
              acc, gi_v, si2d, rows0, rows1, hist, zbuf,
              sem0, sem1):
    c = lax.axis_index("c")
    s = lax.axis_index("s")
    wid = c * NS + s
    base_e = wid * EW

    full = nrows // SLAB
    rem = nrows - full * SLAB
    spt = full // NS if rem == 0 else 1

    zeros16 = jnp.zeros((16,), jnp.float32)
    ones16 = jnp.ones((16,), jnp.float32)

    def _fill(i, _):
        for j in range(D // 16):
            zbuf[i, pl.ds(j * 16, 16)] = zeros16
        return 0
    lax.fori_loop(0, ZR, _fill, 0)

    def _fill_hist(i, _):
        hist[pl.ds(i * 16, 16)] = zeros16
        return 0
    lax.fori_loop(0, nrows // 16, _fill_hist, 0)

    if rem == 0:
        for j in range(spt):
            off = (s * spt + j) * SLAB
            for z in range(SLAB // ZR):
                pltpu.sync_copy(zbuf, acc.at[pl.ds(off + z * ZR, ZR)])
    else:
        @pl.when(s < full)
        def _():
            for z in range(SLAB // ZR):
                pltpu.sync_copy(zbuf, acc.at[pl.ds(s * SLAB + z * ZR, ZR)])

        @pl.when(s == full)
        def _():
            pltpu.sync_copy(zbuf, acc.at[pl.ds(full * SLAB, ZR)])
            pltpu.sync_copy(zbuf.at[pl.ds(0, rem - ZR)],
                            acc.at[pl.ds(full * SLAB + ZR, rem - ZR)])
    plsc.subcore_barrier()

    def _stage(j, rows, semg):
        pltpu.async_copy(table_hbm.at[gi_v.at[pl.ds(j * C, C)]], rows, semg)
        idxs = si2d.at[j]
        for k in range(C // 16):
            idx16 = idxs[pl.ds(k * 16, 16)]
            plsc.addupdate_scatter(hist, [idx16], ones16)

    def _complete(j, rows, semg):
        pltpu.make_async_copy(table_hbm.at[gi_v.at[pl.ds(j * C, C)]],
                              rows, semg).wait()
        pltpu.sync_copy(rows, acc.at[si2d.at[j]], add=True)

    assert G % 2 == 1

    def _super(sc, _):
        pltpu.sync_copy(gidx_hbm.at[pl.ds(base_e + sc * G * C, G * C)], gi_v)
        pltpu.sync_copy(sidx4d_hbm.at[wid, sc], si2d)
        _stage(0, rows0, sem0)

        def _pair(p, _):
            _stage(2 * p + 1, rows1, sem1)
            _complete(2 * p, rows0, sem0)
            _stage(2 * p + 2, rows0, sem0)
            _complete(2 * p + 1, rows1, sem1)
            return 0
        lax.fori_loop(0, (G - 1) // 2, _pair, 0)
        _complete(G - 1, rows0, sem0)
        return 0
    lax.fori_loop(0, NSC, _super, 0)
    plsc.subcore_barrier()

    def _out_piece(off, n):
        pltpu.sync_copy(acc.at[pl.ds(off, n)], zbuf.at[pl.ds(0, n)])
        pltpu.sync_copy(zbuf.at[pl.ds(0, n)],
                        sum_out.at[pl.ds(c * nrows + off, n)])

    if rem == 0:
        for j in range(spt):
            off = (s * spt + j) * SLAB
            for z in range(SLAB // ZR):
                _out_piece(off + z * ZR, ZR)
    else:
        @pl.when(s < full)
        def _():
            for z in range(SLAB // ZR):
                _out_piece(s * SLAB + z * ZR, ZR)

        @pl.when(s == full)
        def _():
            _out_piece(full * SLAB, ZR)
            _out_piece(full * SLAB + ZR, rem - ZR)

    pltpu.sync_copy(hist, cnt_out.at[pl.ds(wid * nrows, nrows)])


def _make_seg(nrows):
    return pl.kernel(
        functools.partial(_seg_body, nrows),
        out_type=[
            jax.ShapeDtypeStruct((NC * nrows, D), jnp.float32),
            jax.ShapeDtypeStruct((NW * nrows,), jnp.float32),
        ],
        mesh=plsc.VectorSubcoreMesh(core_axis_name="c", subcore_axis_name="s"),
        compiler_params=pltpu.CompilerParams(needs_layout_passes=False),
        scratch_types=[
            pltpu.VMEM_SHARED((nrows, D), jnp.float32),
            pltpu.VMEM((G * C,), jnp.int32),
            pltpu.VMEM((G, C), jnp.int32),
            pltpu.VMEM((C, D), jnp.float32),
            pltpu.VMEM((C, D), jnp.float32),
            pltpu.VMEM((nrows,), jnp.float32),
            pltpu.VMEM((ZR, D), jnp.float32),
            pltpu.SemaphoreType.DMA,
            pltpu.SemaphoreType.DMA,
        ],
    )


_seg_trait = _make_seg(NTP)
_seg_bact = _make_seg(NBP)


BLK = 400


def _linear_body(x_ref, w_ref, b_ref, o_ref):
    o_ref[...] = jnp.dot(x_ref[...], w_ref[...],
                         preferred_element_type=jnp.float32) + b_ref[...]


def _linear(x, w, b):
    n = x.shape[0]
    return pl.pallas_call(
        _linear_body,
        grid=(n // BLK,),
        in_specs=[
            pl.BlockSpec((BLK, D), lambda i: (i, 0)),
            pl.BlockSpec((D, D), lambda i: (0, 0)),
            pl.BlockSpec((1, D), lambda i: (0, 0)),
        ],
        out_specs=pl.BlockSpec((BLK, D), lambda i: (i, 0)),
        out_shape=jax.ShapeDtypeStruct((n, D), jnp.float32),
    )(x, w, b.reshape(1, D))


def _post_body(t_ref, s0_ref, s1_ref, c_ref, wl_ref, wr_ref,
               b_ref, g_ref, be_ref, h_ref):
    t = t_ref[...]
    ssum = s0_ref[...] + s1_ref[...]
    cnt = jnp.sum(c_ref[...], axis=-1, keepdims=True)
    mean = ssum / jnp.maximum(cnt, 1.0)
    conv = (jnp.dot(mean, wl_ref[...], preferred_element_type=jnp.float32)
            + jnp.dot(t, wr_ref[...], preferred_element_type=jnp.float32)
            + b_ref[...])
    nrm = jnp.sqrt(jnp.sum(conv * conv, axis=-1, keepdims=True))
    conv = conv / jnp.maximum(nrm, 1e-12)
    x = t + conv
    m = jnp.mean(x, axis=-1, keepdims=True)
    v = jnp.mean((x - m) * (x - m), axis=-1, keepdims=True)
    h_ref[...] = (x - m) / jnp.sqrt(v + 1e-5) * g_ref[...] + be_ref[...]


def _post(t, s0, s1, cmat, wl, wr, b, g, be):
    n = t.shape[0]
    return pl.pallas_call(
        _post_body,
        grid=(n // BLK,),
        in_specs=[
            pl.BlockSpec((BLK, D), lambda i: (i, 0)),
            pl.BlockSpec((BLK, D), lambda i: (i, 0)),
            pl.BlockSpec((BLK, D), lambda i: (i, 0)),
            pl.BlockSpec((BLK, NW), lambda i: (i, 0)),
            pl.BlockSpec((D, D), lambda i: (0, 0)),
            pl.BlockSpec((D, D), lambda i: (0, 0)),
            pl.BlockSpec((1, D), lambda i: (0, 0)),
            pl.BlockSpec((1, D), lambda i: (0, 0)),
            pl.BlockSpec((1, D), lambda i: (0, 0)),
        ],
        out_specs=pl.BlockSpec((BLK, D), lambda i: (i, 0)),
        out_shape=jax.ShapeDtypeStruct((n, D), jnp.float32),
    )(t, s0, s1, cmat, wl, wr, b.reshape(1, D), g.reshape(1, D),
      be.reshape(1, D))


def _agg_body(a1_ref, a2_ref, ht_ref, w_ref, o_ref):
    ht = ht_ref[...]
    a1 = jnp.dot(a1_ref[...], ht, preferred_element_type=jnp.float32)
    a2 = jnp.dot(a2_ref[...], ht, preferred_element_type=jnp.float32)
    wv = w_ref[...]
    mx = jnp.max(wv)
    ev = jnp.exp(wv - mx)
    es = jnp.sum(ev)
    o_ref[...] = (a1 * ev[:, 0:1] + a2 * ev[:, 1:2]) / es


def _agg(adj1, adj2, h_t, mp_w):
    return pl.pallas_call(
        _agg_body,
        grid=(Nb // BLK,),
        in_specs=[
            pl.BlockSpec((BLK, Nt), lambda i: (i, 0)),
            pl.BlockSpec((BLK, Nt), lambda i: (i, 0)),
            pl.BlockSpec((Nt, D), lambda i: (0, 0)),
            pl.BlockSpec((1, 2), lambda i: (0, 0)),
        ],
        out_specs=pl.BlockSpec((BLK, D), lambda i: (i, 0)),
        out_shape=jax.ShapeDtypeStruct((Nb, D), jnp.float32),
    )(adj1, adj2, h_t, mp_w.reshape(1, 2))


def _out_body(t_ref, s0_ref, s1_ref, c_ref, agg_ref, wl_ref, wr_ref, b_ref,
              g_ref, be_ref, wmp_ref, bmp_ref, gmp_ref, bemp_ref, o_ref):
    t = t_ref[...]
    ssum = s0_ref[...] + s1_ref[...]
    cnt = jnp.sum(c_ref[...], axis=-1, keepdims=True)
    mean = ssum / jnp.maximum(cnt, 1.0)
    conv = (jnp.dot(mean, wl_ref[...], preferred_element_type=jnp.float32)
            + jnp.dot(t, wr_ref[...], preferred_element_type=jnp.float32)
            + b_ref[...])
    nrm = jnp.sqrt(jnp.sum(conv * conv, axis=-1, keepdims=True))
    conv = conv / jnp.maximum(nrm, 1e-12)
    x = t + conv
    m = jnp.mean(x, axis=-1, keepdims=True)
    v = jnp.mean((x - m) * (x - m), axis=-1, keepdims=True)
    h_b = (x - m) / jnp.sqrt(v + 1e-5) * g_ref[...] + be_ref[...]
    y = h_b + agg_ref[...]
    z = (jnp.dot(y, wmp_ref[...], preferred_element_type=jnp.float32)
         + bmp_ref[...])
    m2 = jnp.mean(z, axis=-1, keepdims=True)
    v2 = jnp.mean((z - m2) * (z - m2), axis=-1, keepdims=True)
    o_ref[...] = (z - m2) / jnp.sqrt(v2 + 1e-5) * gmp_ref[...] + bemp_ref[...]


def _out(tb, s0, s1, cmat, agg, wl, wr, b, g, be, wmp, bmp, gmp, bemp):
    vec = pl.BlockSpec((1, D), lambda i: (0, 0))
    return pl.pallas_call(
        _out_body,
        grid=(Nb // BLK,),
        in_specs=[
            pl.BlockSpec((BLK, D), lambda i: (i, 0)),
            pl.BlockSpec((BLK, D), lambda i: (i, 0)),
            pl.BlockSpec((BLK, D), lambda i: (i, 0)),
            pl.BlockSpec((BLK, NW), lambda i: (i, 0)),
            pl.BlockSpec((BLK, D), lambda i: (i, 0)),
            pl.BlockSpec((D, D), lambda i: (0, 0)),
            pl.BlockSpec((D, D), lambda i: (0, 0)),
            vec, vec, vec,
            pl.BlockSpec((D, D), lambda i: (0, 0)),
            vec, vec, vec,
        ],
        out_specs=pl.BlockSpec((BLK, D), lambda i: (i, 0)),
        out_shape=jax.ShapeDtypeStruct((Nb, D), jnp.float32),
    )(tb, s0, s1, cmat, agg, wl, wr, b.reshape(1, D), g.reshape(1, D),
      be.reshape(1, D), wmp, bmp.reshape(1, D), gmp.reshape(1, D),
      bemp.reshape(1, D))


def kernel(x_bacteria, x_trait, adj1, adj2, Wb, bb, Wt, bt, Wl_bt, Wr_bt,
           b_bt, Wl_tb, Wr_tb, b_tb, g_b, be_b, g_t, be_t, mp_w, Wmp, bmp,
           g_mp, be_mp, edge_src, edge_dst):
    tb = _linear(x_bacteria, Wb, bb)
    tt = _linear(x_trait, Wt, bt)

    src4d = edge_src.reshape(NW, NSC, G, C)
    dst4d = edge_dst.reshape(NW, NSC, G, C)
    sum_bt, cnt_bt = _seg_trait(tb, edge_src, dst4d)

    h_t = _post(tt, sum_bt[:Nt], sum_bt[NTP:NTP + Nt],
                cnt_bt.reshape(NW, NTP).T[:Nt],
                Wl_bt, Wr_bt, b_bt, g_t, be_t)
    agg = _agg(adj1, adj2, h_t, mp_w)
    sum_tb, cnt_tb = _seg_bact(tt, edge_dst, src4d)

    return _out(tb, sum_tb[:Nb], sum_tb[NBP:NBP + Nb],
                cnt_tb.reshape(NW, NBP).T[:Nb], agg,
                Wl_tb, Wr_tb, b_tb, g_b, be_b, Wmp, bmp, g_mp, be_mp)


# speedup vs baseline: 1.1427x; 1.0423x over previous
import jax
import jax.numpy as jnp
from jax import lax
from jax.experimental import pallas as pl
from jax.experimental.pallas import tpu as pltpu
from jax.experimental.pallas import tpu_sc as plsc
import functools

Nb, Nt, E, D = 10000, 2000, 320000, 128
NC, NS = 2, 16
NW = NC * NS
C = 80
EW = E // NW
CH = EW // C
SLAB = 128
G = 25
NSC = CH // G
ZR = 64
NTP = Nt
TREM = Nt - (NS - 1) * SLAB
NBP = NS * 5 * SLAB


def _seg_body(nrows, table_hbm, gidx_hbm, sidx4d_hbm, sum_out, cnt_out,
              acc, gi_v, si2d, rows0, rows1, hist, zbuf,
              sem0, sem1):
    c = lax.axis_index("c")
    s = lax.axis_index("s")
    wid = c * NS + s
    base_e = wid * EW

    full = nrows // SLAB
    rem = nrows - full * SLAB
    spt = full // NS if rem == 0 else 1

    zeros16 = jnp.zeros((16,), jnp.float32)
    ones16 = jnp.ones((16,), jnp.float32)

    def _fill(i, _):
        for j in range(D // 16):
            zbuf[i, pl.ds(j * 16, 16)] = zeros16
        return 0
    lax.fori_loop(0, ZR, _fill, 0)

    def _fill_hist(i, _):
        hist[pl.ds(i * 16, 16)] = zeros16
        return 0
    lax.fori_loop(0, nrows // 16, _fill_hist, 0)

    if rem == 0:
        for j in range(spt):
            off = (s * spt + j) * SLAB
            for z in range(SLAB // ZR):
                pltpu.sync_copy(zbuf, acc.at[pl.ds(off + z * ZR, ZR)])
    else:
        @pl.when(s < full)
        def _():
            for z in range(SLAB // ZR):
                pltpu.sync_copy(zbuf, acc.at[pl.ds(s * SLAB + z * ZR, ZR)])

        @pl.when(s == full)
        def _():
            pltpu.sync_copy(zbuf, acc.at[pl.ds(full * SLAB, ZR)])
            pltpu.sync_copy(zbuf.at[pl.ds(0, rem - ZR)],
                            acc.at[pl.ds(full * SLAB + ZR, rem - ZR)])
    plsc.subcore_barrier()

    def _stage(j, rows, semg):
        pltpu.async_copy(table_hbm.at[gi_v.at[pl.ds(j * C, C)]], rows, semg)
        idxs = si2d.at[j]
        for k in range(C // 16):
            idx16 = idxs[pl.ds(k * 16, 16)]
            plsc.addupdate_scatter(hist, [idx16], ones16)

    def _complete(j, rows, semg):
        pltpu.make_async_copy(table_hbm.at[gi_v.at[pl.ds(j * C, C)]],
                              rows, semg).wait()
        pltpu.sync_copy(rows, acc.at[si2d.at[j]], add=True)

    assert G % 2 == 1

    def _super(sc, _):
        pltpu.sync_copy(gidx_hbm.at[pl.ds(base_e + sc * G * C, G * C)], gi_v)
        pltpu.sync_copy(sidx4d_hbm.at[wid, sc], si2d)
        _stage(0, rows0, sem0)

        def _pair(p, _):
            _stage(2 * p + 1, rows1, sem1)
            _complete(2 * p, rows0, sem0)
            _stage(2 * p + 2, rows0, sem0)
            _complete(2 * p + 1, rows1, sem1)
            return 0
        lax.fori_loop(0, (G - 1) // 2, _pair, 0)
        _complete(G - 1, rows0, sem0)
        return 0
    lax.fori_loop(0, NSC, _super, 0)
    plsc.subcore_barrier()

    def _out_piece(off, n):
        pltpu.sync_copy(acc.at[pl.ds(off, n)], zbuf.at[pl.ds(0, n)])
        pltpu.sync_copy(zbuf.at[pl.ds(0, n)],
                        sum_out.at[pl.ds(c * nrows + off, n)])

    if rem == 0:
        for j in range(spt):
            off = (s * spt + j) * SLAB
            for z in range(SLAB // ZR):
                _out_piece(off + z * ZR, ZR)
    else:
        @pl.when(s < full)
        def _():
            for z in range(SLAB // ZR):
                _out_piece(s * SLAB + z * ZR, ZR)

        @pl.when(s == full)
        def _():
            _out_piece(full * SLAB, ZR)
            _out_piece(full * SLAB + ZR, rem - ZR)

    pltpu.sync_copy(hist, cnt_out.at[pl.ds(wid * nrows, nrows)])


def _make_seg(nrows):
    return pl.kernel(
        functools.partial(_seg_body, nrows),
        out_type=[
            jax.ShapeDtypeStruct((NC * nrows, D), jnp.float32),
            jax.ShapeDtypeStruct((NW * nrows,), jnp.float32),
        ],
        mesh=plsc.VectorSubcoreMesh(core_axis_name="c", subcore_axis_name="s"),
        compiler_params=pltpu.CompilerParams(needs_layout_passes=False),
        scratch_types=[
            pltpu.VMEM_SHARED((nrows, D), jnp.float32),
            pltpu.VMEM((G * C,), jnp.int32),
            pltpu.VMEM((G, C), jnp.int32),
            pltpu.VMEM((C, D), jnp.float32),
            pltpu.VMEM((C, D), jnp.float32),
            pltpu.VMEM((nrows,), jnp.float32),
            pltpu.VMEM((ZR, D), jnp.float32),
            pltpu.SemaphoreType.DMA,
            pltpu.SemaphoreType.DMA,
        ],
    )


_seg_trait = _make_seg(NTP)
_seg_bact = _make_seg(NBP)


BLK = 400


def _linear_body(x_ref, w_ref, b_ref, o_ref):
    o_ref[...] = jnp.dot(x_ref[...], w_ref[...],
                         preferred_element_type=jnp.float32) + b_ref[...]


def _linear(x, w, b):
    n = x.shape[0]
    return pl.pallas_call(
        _linear_body,
        grid=(n // BLK,),
        in_specs=[
            pl.BlockSpec((BLK, D), lambda i: (i, 0)),
            pl.BlockSpec((D, D), lambda i: (0, 0)),
            pl.BlockSpec((1, D), lambda i: (0, 0)),
        ],
        out_specs=pl.BlockSpec((BLK, D), lambda i: (i, 0)),
        out_shape=jax.ShapeDtypeStruct((n, D), jnp.float32),
    )(x, w, b.reshape(1, D))


def _post_body(t_ref, s0_ref, s1_ref, c_ref, wl_ref, wr_ref,
               b_ref, g_ref, be_ref, h_ref):
    t = t_ref[...]
    ssum = s0_ref[...] + s1_ref[...]
    cnt = jnp.sum(c_ref[...], axis=-1, keepdims=True)
    mean = ssum / jnp.maximum(cnt, 1.0)
    conv = (jnp.dot(mean, wl_ref[...], preferred_element_type=jnp.float32)
            + jnp.dot(t, wr_ref[...], preferred_element_type=jnp.float32)
            + b_ref[...])
    nrm = jnp.sqrt(jnp.sum(conv * conv, axis=-1, keepdims=True))
    conv = conv / jnp.maximum(nrm, 1e-12)
    x = t + conv
    m = jnp.mean(x, axis=-1, keepdims=True)
    v = jnp.mean((x - m) * (x - m), axis=-1, keepdims=True)
    h_ref[...] = (x - m) / jnp.sqrt(v + 1e-5) * g_ref[...] + be_ref[...]


def _post(t, s0, s1, cmat, wl, wr, b, g, be):
    n = t.shape[0]
    return pl.pallas_call(
        _post_body,
        grid=(n // BLK,),
        in_specs=[
            pl.BlockSpec((BLK, D), lambda i: (i, 0)),
            pl.BlockSpec((BLK, D), lambda i: (i, 0)),
            pl.BlockSpec((BLK, D), lambda i: (i, 0)),
            pl.BlockSpec((BLK, NW), lambda i: (i, 0)),
            pl.BlockSpec((D, D), lambda i: (0, 0)),
            pl.BlockSpec((D, D), lambda i: (0, 0)),
            pl.BlockSpec((1, D), lambda i: (0, 0)),
            pl.BlockSpec((1, D), lambda i: (0, 0)),
            pl.BlockSpec((1, D), lambda i: (0, 0)),
        ],
        out_specs=pl.BlockSpec((BLK, D), lambda i: (i, 0)),
        out_shape=jax.ShapeDtypeStruct((n, D), jnp.float32),
    )(t, s0, s1, cmat, wl, wr, b.reshape(1, D), g.reshape(1, D),
      be.reshape(1, D))


def _aggout_body(a1_hbm, a2_hbm, ht_ref, w_ref, t_ref, s0_ref, s1_ref, c_ref,
                 wl_ref, wr_ref, b_ref, g_ref, be_ref, wmp_ref, bmp_ref,
                 gmp_ref, bemp_ref, o_ref, a1v, a2v, sems):
    i = pl.program_id(0)
    n = pl.num_programs(0)
    cur = lax.rem(i, 2)
    nxt = lax.rem(i + 1, 2)

    @pl.when(i == 0)
    def _():
        pltpu.async_copy(a1_hbm.at[pl.ds(0, BLK)], a1v.at[0], sems.at[0])
        pltpu.async_copy(a2_hbm.at[pl.ds(0, BLK)], a2v.at[0], sems.at[0])

    @pl.when(i + 1 < n)
    def _():
        pltpu.async_copy(a1_hbm.at[pl.ds((i + 1) * BLK, BLK)], a1v.at[nxt],
                         sems.at[nxt])
        pltpu.async_copy(a2_hbm.at[pl.ds((i + 1) * BLK, BLK)], a2v.at[nxt],
                         sems.at[nxt])

    pltpu.make_async_copy(a1_hbm.at[pl.ds(i * BLK, BLK)], a1v.at[cur],
                          sems.at[cur]).wait()
    pltpu.make_async_copy(a2_hbm.at[pl.ds(i * BLK, BLK)], a2v.at[cur],
                          sems.at[cur]).wait()

    ht = ht_ref[...]
    a1 = jnp.dot(a1v[cur], ht, preferred_element_type=jnp.float32)
    a2 = jnp.dot(a2v[cur], ht, preferred_element_type=jnp.float32)
    wv = w_ref[...]
    ev = jnp.exp(wv - jnp.max(wv))
    es = jnp.sum(ev)
    agg = (a1 * ev[:, 0:1] + a2 * ev[:, 1:2]) / es

    t = t_ref[...]
    ssum = s0_ref[...] + s1_ref[...]
    cnt = jnp.sum(c_ref[...], axis=-1, keepdims=True)
    mean = ssum / jnp.maximum(cnt, 1.0)
    conv = (jnp.dot(mean, wl_ref[...], preferred_element_type=jnp.float32)
            + jnp.dot(t, wr_ref[...], preferred_element_type=jnp.float32)
            + b_ref[...])
    nrm = jnp.sqrt(jnp.sum(conv * conv, axis=-1, keepdims=True))
    conv = conv / jnp.maximum(nrm, 1e-12)
    x = t + conv
    m = jnp.mean(x, axis=-1, keepdims=True)
    v = jnp.mean((x - m) * (x - m), axis=-1, keepdims=True)
    h_b = (x - m) / jnp.sqrt(v + 1e-5) * g_ref[...] + be_ref[...]
    y = h_b + agg
    z = (jnp.dot(y, wmp_ref[...], preferred_element_type=jnp.float32)
         + bmp_ref[...])
    m2 = jnp.mean(z, axis=-1, keepdims=True)
    v2 = jnp.mean((z - m2) * (z - m2), axis=-1, keepdims=True)
    o_ref[...] = (z - m2) / jnp.sqrt(v2 + 1e-5) * gmp_ref[...] + bemp_ref[...]


def _aggout(adj1, adj2, h_t, mp_w, tb, s0, s1, cmat,
            wl, wr, b, g, be, wmp, bmp, gmp, bemp):
    vec = pl.BlockSpec((1, D), lambda i: (0, 0))
    mat = pl.BlockSpec((D, D), lambda i: (0, 0))
    blk = pl.BlockSpec((BLK, D), lambda i: (i, 0))
    return pl.pallas_call(
        _aggout_body,
        grid=(Nb // BLK,),
        in_specs=[
            pl.BlockSpec(memory_space=pl.ANY),
            pl.BlockSpec(memory_space=pl.ANY),
            pl.BlockSpec((Nt, D), lambda i: (0, 0)),
            pl.BlockSpec((1, 2), lambda i: (0, 0)),
            blk, blk, blk,
            pl.BlockSpec((BLK, NW), lambda i: (i, 0)),
            mat, mat, vec, vec, vec, mat, vec, vec, vec,
        ],
        out_specs=blk,
        out_shape=jax.ShapeDtypeStruct((Nb, D), jnp.float32),
        scratch_shapes=[
            pltpu.VMEM((2, BLK, Nt), jnp.float32),
            pltpu.VMEM((2, BLK, Nt), jnp.float32),
            pltpu.SemaphoreType.DMA((2,)),
        ],
    )(adj1, adj2, h_t, mp_w.reshape(1, 2), tb, s0, s1, cmat,
      wl, wr, b.reshape(1, D), g.reshape(1, D), be.reshape(1, D),
      wmp, bmp.reshape(1, D), gmp.reshape(1, D), bemp.reshape(1, D))


def kernel(x_bacteria, x_trait, adj1, adj2, Wb, bb, Wt, bt, Wl_bt, Wr_bt,
           b_bt, Wl_tb, Wr_tb, b_tb, g_b, be_b, g_t, be_t, mp_w, Wmp, bmp,
           g_mp, be_mp, edge_src, edge_dst):
    tb = _linear(x_bacteria, Wb, bb)
    tt = _linear(x_trait, Wt, bt)

    src4d = edge_src.reshape(NW, NSC, G, C)
    dst4d = edge_dst.reshape(NW, NSC, G, C)
    sum_bt, cnt_bt = _seg_trait(tb, edge_src, dst4d)

    h_t = _post(tt, sum_bt[:Nt], sum_bt[NTP:NTP + Nt],
                cnt_bt.reshape(NW, NTP).T[:Nt],
                Wl_bt, Wr_bt, b_bt, g_t, be_t)
    sum_tb, cnt_tb = _seg_bact(tt, edge_dst, src4d)

    return _aggout(adj1, adj2, h_t, mp_w, tb,
                   sum_tb[:Nb], sum_tb[NBP:NBP + Nb],
                   cnt_tb.reshape(NW, NBP).T[:Nb],
                   Wl_tb, Wr_tb, b_tb, g_b, be_b, Wmp, bmp, g_mp, be_mp)


# speedup vs baseline: 1.2142x; 1.0626x over previous
import jax
import jax.numpy as jnp
from jax import lax
from jax.experimental import pallas as pl
from jax.experimental.pallas import tpu as pltpu
from jax.experimental.pallas import tpu_sc as plsc
import functools

Nb, Nt, E, D = 10000, 2000, 320000, 128
NC, NS = 2, 16
NW = NC * NS
C = 80
EW = E // NW
CH = EW // C
SLAB = 128
G = 25
NSC = CH // G
ZR = 64
NTP = Nt
TREM = Nt - (NS - 1) * SLAB
NBP = NS * 5 * SLAB


def _seg_body(nrows, table_hbm, gidx_hbm, sidx4d_hbm, sum_out, cnt_out,
              acc, gi_v, si2d, rows0, rows1, hist, zbuf,
              sem0, sem1):
    c = lax.axis_index("c")
    s = lax.axis_index("s")
    wid = c * NS + s
    base_e = wid * EW

    full = nrows // SLAB
    rem = nrows - full * SLAB
    spt = full // NS if rem == 0 else 1

    zeros16 = jnp.zeros((16,), jnp.float32)
    ones16 = jnp.ones((16,), jnp.float32)

    def _fill(i, _):
        for j in range(D // 16):
            zbuf[i, pl.ds(j * 16, 16)] = zeros16
        return 0
    lax.fori_loop(0, ZR, _fill, 0)

    def _fill_hist(i, _):
        hist[pl.ds(i * 16, 16)] = zeros16
        return 0
    lax.fori_loop(0, nrows // 16, _fill_hist, 0)

    if rem == 0:
        for j in range(spt):
            off = (s * spt + j) * SLAB
            for z in range(SLAB // ZR):
                pltpu.sync_copy(zbuf, acc.at[pl.ds(off + z * ZR, ZR)])
    else:
        @pl.when(s < full)
        def _():
            for z in range(SLAB // ZR):
                pltpu.sync_copy(zbuf, acc.at[pl.ds(s * SLAB + z * ZR, ZR)])

        @pl.when(s == full)
        def _():
            pltpu.sync_copy(zbuf, acc.at[pl.ds(full * SLAB, ZR)])
            pltpu.sync_copy(zbuf.at[pl.ds(0, rem - ZR)],
                            acc.at[pl.ds(full * SLAB + ZR, rem - ZR)])
    plsc.subcore_barrier()

    def _stage(j, rows, semg):
        pltpu.async_copy(table_hbm.at[gi_v.at[pl.ds(j * C, C)]], rows, semg)
        idxs = si2d.at[j]
        for k in range(C // 16):
            idx16 = idxs[pl.ds(k * 16, 16)]
            plsc.addupdate_scatter(hist, [idx16], ones16)

    def _complete(j, rows, semg):
        pltpu.make_async_copy(table_hbm.at[gi_v.at[pl.ds(j * C, C)]],
                              rows, semg).wait()
        pltpu.sync_copy(rows, acc.at[si2d.at[j]], add=True)

    assert G % 2 == 1

    def _super(sc, _):
        pltpu.sync_copy(gidx_hbm.at[pl.ds(base_e + sc * G * C, G * C)], gi_v)
        pltpu.sync_copy(sidx4d_hbm.at[wid, sc], si2d)
        _stage(0, rows0, sem0)

        def _pair(p, _):
            _stage(2 * p + 1, rows1, sem1)
            _complete(2 * p, rows0, sem0)
            _stage(2 * p + 2, rows0, sem0)
            _complete(2 * p + 1, rows1, sem1)
            return 0
        lax.fori_loop(0, (G - 1) // 2, _pair, 0)
        _complete(G - 1, rows0, sem0)
        return 0
    lax.fori_loop(0, NSC, _super, 0)
    plsc.subcore_barrier()

    def _out_piece(off, n):
        pltpu.sync_copy(acc.at[pl.ds(off, n)], zbuf.at[pl.ds(0, n)])
        pltpu.sync_copy(zbuf.at[pl.ds(0, n)],
                        sum_out.at[pl.ds(c * nrows + off, n)])

    if rem == 0:
        for j in range(spt):
            off = (s * spt + j) * SLAB
            for z in range(SLAB // ZR):
                _out_piece(off + z * ZR, ZR)
    else:
        @pl.when(s < full)
        def _():
            for z in range(SLAB // ZR):
                _out_piece(s * SLAB + z * ZR, ZR)

        @pl.when(s == full)
        def _():
            _out_piece(full * SLAB, ZR)
            _out_piece(full * SLAB + ZR, rem - ZR)

    pltpu.sync_copy(hist, cnt_out.at[pl.ds(wid * nrows, nrows)])


def _make_seg(nrows):
    return pl.kernel(
        functools.partial(_seg_body, nrows),
        out_type=[
            jax.ShapeDtypeStruct((NC * nrows, D), jnp.float32),
            jax.ShapeDtypeStruct((NW * nrows,), jnp.float32),
        ],
        mesh=plsc.VectorSubcoreMesh(core_axis_name="c", subcore_axis_name="s"),
        compiler_params=pltpu.CompilerParams(needs_layout_passes=False),
        scratch_types=[
            pltpu.VMEM_SHARED((nrows, D), jnp.float32),
            pltpu.VMEM((G * C,), jnp.int32),
            pltpu.VMEM((G, C), jnp.int32),
            pltpu.VMEM((C, D), jnp.float32),
            pltpu.VMEM((C, D), jnp.float32),
            pltpu.VMEM((nrows,), jnp.float32),
            pltpu.VMEM((ZR, D), jnp.float32),
            pltpu.SemaphoreType.DMA,
            pltpu.SemaphoreType.DMA,
        ],
    )


_seg_trait = _make_seg(NTP)
_seg_bact = _make_seg(NBP)


BLK = 400


def _linear_body(x_ref, w_ref, b_ref, o_ref):
    o_ref[...] = jnp.dot(x_ref[...], w_ref[...],
                         preferred_element_type=jnp.float32) + b_ref[...]


def _linear(x, w, b):
    n = x.shape[0]
    blkl = 2000
    return pl.pallas_call(
        _linear_body,
        grid=(n // blkl,),
        in_specs=[
            pl.BlockSpec((blkl, D), lambda i: (i, 0)),
            pl.BlockSpec((D, D), lambda i: (0, 0)),
            pl.BlockSpec((1, D), lambda i: (0, 0)),
        ],
        out_specs=pl.BlockSpec((blkl, D), lambda i: (i, 0)),
        out_shape=jax.ShapeDtypeStruct((n, D), jnp.float32),
    )(x, w, b.reshape(1, D))


def _sage_norm(t, s0, s1, cnt, wl_ref, wr_ref, b_ref, g_ref, be_ref):
    mean = (s0 + s1) / jnp.maximum(cnt, 1.0)
    conv = (jnp.dot(mean, wl_ref[...], preferred_element_type=jnp.float32)
            + jnp.dot(t, wr_ref[...], preferred_element_type=jnp.float32)
            + b_ref[...])
    nrm = jnp.sqrt(jnp.sum(conv * conv, axis=-1, keepdims=True))
    conv = conv / jnp.maximum(nrm, 1e-12)
    x = t + conv
    m = jnp.mean(x, axis=-1, keepdims=True)
    v = jnp.mean((x - m) * (x - m), axis=-1, keepdims=True)
    return (x - m) / jnp.sqrt(v + 1e-5) * g_ref[...] + be_ref[...]


def _aggout_body(a1_hbm, a2_hbm, w_ref,
                 tt_ref, sbt0_ref, sbt1_ref, cbt_ref,
                 wlt_ref, wrt_ref, bt_ref, gt_ref, bet_ref,
                 t_ref, s0_ref, s1_ref, c_ref,
                 wl_ref, wr_ref, b_ref, g_ref, be_ref,
                 wmp_ref, bmp_ref, gmp_ref, bemp_ref,
                 o_ref, a1v, a2v, ht_s, sems):
    i = pl.program_id(0)
    n = pl.num_programs(0)
    cur = lax.rem(i, 2)
    nxt = lax.rem(i + 1, 2)

    @pl.when(i == 0)
    def _():
        pltpu.async_copy(a1_hbm.at[pl.ds(0, BLK)], a1v.at[0], sems.at[0])
        pltpu.async_copy(a2_hbm.at[pl.ds(0, BLK)], a2v.at[0], sems.at[0])
        cbt = jnp.sum(cbt_ref[...], axis=-1, keepdims=True)
        ht_s[...] = _sage_norm(tt_ref[...], sbt0_ref[0], sbt1_ref[0], cbt,
                               wlt_ref, wrt_ref, bt_ref, gt_ref, bet_ref)

    @pl.when(i + 1 < n)
    def _():
        pltpu.async_copy(a1_hbm.at[pl.ds((i + 1) * BLK, BLK)], a1v.at[nxt],
                         sems.at[nxt])
        pltpu.async_copy(a2_hbm.at[pl.ds((i + 1) * BLK, BLK)], a2v.at[nxt],
                         sems.at[nxt])

    pltpu.make_async_copy(a1_hbm.at[pl.ds(i * BLK, BLK)], a1v.at[cur],
                          sems.at[cur]).wait()
    pltpu.make_async_copy(a2_hbm.at[pl.ds(i * BLK, BLK)], a2v.at[cur],
                          sems.at[cur]).wait()

    ht = ht_s[...]
    a1 = jnp.dot(a1v[cur], ht, preferred_element_type=jnp.float32)
    a2 = jnp.dot(a2v[cur], ht, preferred_element_type=jnp.float32)
    wv = w_ref[...]
    ev = jnp.exp(wv - jnp.max(wv))
    es = jnp.sum(ev)
    agg = (a1 * ev[:, 0:1] + a2 * ev[:, 1:2]) / es

    cnt = jnp.sum(c_ref[...], axis=-1, keepdims=True)
    h_b = _sage_norm(t_ref[...], s0_ref[0], s1_ref[0], cnt,
                     wl_ref, wr_ref, b_ref, g_ref, be_ref)
    y = h_b + agg
    z = (jnp.dot(y, wmp_ref[...], preferred_element_type=jnp.float32)
         + bmp_ref[...])
    m2 = jnp.mean(z, axis=-1, keepdims=True)
    v2 = jnp.mean((z - m2) * (z - m2), axis=-1, keepdims=True)
    o_ref[...] = (z - m2) / jnp.sqrt(v2 + 1e-5) * gmp_ref[...] + bemp_ref[...]


def _aggout(adj1, adj2, mp_w,
            tt, sum_bt, cnt_btT, wlt, wrt, bt2, gt, bet,
            tb, sum_tb, cnt_tbT, wl, wr, b, g, be, wmp, bmp, gmp, bemp):
    vec = pl.BlockSpec((1, D), lambda i: (0, 0))
    mat = pl.BlockSpec((D, D), lambda i: (0, 0))
    blk = pl.BlockSpec((BLK, D), lambda i: (i, 0))
    return pl.pallas_call(
        _aggout_body,
        grid=(Nb // BLK,),
        in_specs=[
            pl.BlockSpec(memory_space=pl.ANY),
            pl.BlockSpec(memory_space=pl.ANY),
            pl.BlockSpec((1, 2), lambda i: (0, 0)),
            pl.BlockSpec((Nt, D), lambda i: (0, 0)),
            pl.BlockSpec((1, Nt, D), lambda i: (0, 0, 0)),
            pl.BlockSpec((1, Nt, D), lambda i: (1, 0, 0)),
            pl.BlockSpec((Nt, NW), lambda i: (0, 0)),
            mat, mat, vec, vec, vec,
            blk,
            pl.BlockSpec((1, BLK, D), lambda i: (0, i, 0)),
            pl.BlockSpec((1, BLK, D), lambda i: (1, i, 0)),
            pl.BlockSpec((BLK, NW), lambda i: (i, 0)),
            mat, mat, vec, vec, vec, mat, vec, vec, vec,
        ],
        out_specs=blk,
        out_shape=jax.ShapeDtypeStruct((Nb, D), jnp.float32),
        scratch_shapes=[
            pltpu.VMEM((2, BLK, Nt), jnp.float32),
            pltpu.VMEM((2, BLK, Nt), jnp.float32),
            pltpu.VMEM((Nt, D), jnp.float32),
            pltpu.SemaphoreType.DMA((2,)),
        ],
    )(adj1, adj2, mp_w.reshape(1, 2),
      tt, sum_bt, sum_bt, cnt_btT, wlt, wrt, bt2.reshape(1, D),
      gt.reshape(1, D), bet.reshape(1, D),
      tb, sum_tb, sum_tb, cnt_tbT, wl, wr, b.reshape(1, D), g.reshape(1, D),
      be.reshape(1, D), wmp, bmp.reshape(1, D), gmp.reshape(1, D),
      bemp.reshape(1, D))


def kernel(x_bacteria, x_trait, adj1, adj2, Wb, bb, Wt, bt, Wl_bt, Wr_bt,
           b_bt, Wl_tb, Wr_tb, b_tb, g_b, be_b, g_t, be_t, mp_w, Wmp, bmp,
           g_mp, be_mp, edge_src, edge_dst):
    tb = _linear(x_bacteria, Wb, bb)
    tt = _linear(x_trait, Wt, bt)

    src4d = edge_src.reshape(NW, NSC, G, C)
    dst4d = edge_dst.reshape(NW, NSC, G, C)
    sum_bt, cnt_bt = _seg_trait(tb, edge_src, dst4d)
    sum_tb, cnt_tb = _seg_bact(tt, edge_dst, src4d)

    return _aggout(adj1, adj2, mp_w,
                   tt, sum_bt.reshape(2, NTP, D), cnt_bt.reshape(NW, NTP).T,
                   Wl_bt, Wr_bt, b_bt, g_t, be_t,
                   tb, sum_tb.reshape(2, NBP, D), cnt_tb.reshape(NW, NBP).T,
                   Wl_tb, Wr_tb, b_tb, g_b, be_b, Wmp, bmp, g_mp, be_mp)
